# Initial kernel scaffold; baseline (speedup 1.0000x reference)
import jax
import jax.numpy as jnp
from jax.experimental import pallas as pl


def kernel(features, W0, b0, W1, b1, sample0, sample1, sample2, block0_src, block0_dst, block1_src, block1_dst):
    raise NotImplementedError("write your pallas kernel here")


# speedup vs baseline: 2.6126x; 2.6126x over previous
import functools

import jax
import jax.numpy as jnp
from jax import lax
from jax.experimental import pallas as pl
from jax.experimental.pallas import tpu as pltpu
from jax.experimental.pallas import tpu_sc as plsc

_NC, _NS, _L = 2, 16, 16
_NW = _NC * _NS

_D = 128
_N0, _N1, _N2 = 2048, 32768, 524288
_E0, _E1 = 32768, 524288
_B = 128


def _mesh():
    return plsc.VectorSubcoreMesh(core_axis_name="c", subcore_axis_name="s")


def _params():
    return pltpu.CompilerParams(needs_layout_passes=False)


def _zero_zbuf(zbuf):
    z = jnp.zeros((_L,), jnp.float32)
    for r in range(16):
        for v in range(_D // _L):
            zbuf[r, pl.ds(v * _L, _L)] = z


def _stage_body(src_hbm, s2_hbm, s1_hbm, s0_hbm, feat_hbm,
                idx1_hbm, g1_hbm, g0_hbm,
                srcv, idx1v, s1v, s0v, rows_a, rows_b, g0rows, sem):
    wid = lax.axis_index("s") * _NC + lax.axis_index("c")
    epw = _E1 // _NW
    ebase = wid * epw
    pltpu.sync_copy(src_hbm.at[pl.ds(ebase, epw)], srcv)

    def compose_group(g, _):
        descs = []
        for b in range(8):
            off = (g * 8 + b) * _B
            descs.append(pltpu.async_copy(
                s2_hbm.at[srcv.at[pl.ds(off, _B)]],
                idx1v.at[pl.ds(off, _B)], sem))
        for d in descs:
            d.wait()
        return _
    lax.fori_loop(0, epw // _B // 8, compose_group, 0)
    pltpu.sync_copy(idx1v, idx1_hbm.at[pl.ds(ebase, epw)])

    npw = _N1 // _NW
    nbase = wid * npw
    pltpu.sync_copy(s1_hbm.at[pl.ds(nbase, npw)], s1v)

    def g1_group(g, _):
        j0 = g * 2
        d0 = pltpu.async_copy(
            feat_hbm.at[s1v.at[pl.ds(j0 * _B, _B)]], rows_a, sem)
        d1 = pltpu.async_copy(
            feat_hbm.at[s1v.at[pl.ds((j0 + 1) * _B, _B)]], rows_b, sem)
        d0.wait()
        pltpu.sync_copy(rows_a, g1_hbm.at[pl.ds(nbase + j0 * _B, _B), :])
        d1.wait()
        pltpu.sync_copy(rows_b, g1_hbm.at[pl.ds(nbase + (j0 + 1) * _B, _B), :])
        return _
    lax.fori_loop(0, npw // _B // 2, g1_group, 0)

    zpw = _N0 // _NW
    zbase = wid * zpw
    pltpu.sync_copy(s0_hbm.at[pl.ds(zbase, zpw)], s0v)
    pltpu.async_copy(feat_hbm.at[s0v], g0rows, sem).wait()
    pltpu.sync_copy(g0rows, g0_hbm.at[pl.ds(zbase, zpw), :])


def _stage(block1_src, sample2, sample1, sample0, features):
    f = pl.kernel(
        _stage_body,
        out_type=(
            jax.ShapeDtypeStruct((_E1,), jnp.int32),
            jax.ShapeDtypeStruct((_N1, _D), jnp.float32),
            jax.ShapeDtypeStruct((_N0, _D), jnp.float32),
        ),
        mesh=_mesh(),
        compiler_params=_params(),
        scratch_types=[
            pltpu.VMEM((_E1 // _NW,), jnp.int32),
            pltpu.VMEM((_E1 // _NW,), jnp.int32),
            pltpu.VMEM((_N1 // _NW,), jnp.int32),
            pltpu.VMEM((_N0 // _NW,), jnp.int32),
            pltpu.VMEM((_B, _D), jnp.float32),
            pltpu.VMEM((_B, _D), jnp.float32),
            pltpu.VMEM((_N0 // _NW, _D), jnp.float32),
            pltpu.SemaphoreType.DMA,
        ],
        name="sage_stage",
    )
    return f(block1_src, sample2, sample1, sample0, features)


def _make_agg(e_tot, ndst, chunk, name):
    nch = ndst // chunk // _NC
    ept = e_tot // _NS
    slabsz = min(4096, ept)
    trash = ept + _B - 1
    spt = chunk // _NS

    def body(feat_hbm, idx_hbm, dst_hbm,
             acc_hbm, cnt_hbm,
             accsh, cntsh, dstv, idxv, cidx, cdst, rows,
             cntloc, redbuf, sumv, zbuf, sem):
        cid = lax.axis_index("c")
        sid = lax.axis_index("s")
        ones = jnp.ones((_L,), jnp.float32)
        _zero_zbuf(zbuf)

        def chunk_body(kk, carry):
            lo = (nch * cid + kk) * chunk

            descs = []
            for r in range(spt // 16):
                descs.append(pltpu.async_copy(
                    zbuf, accsh.at[pl.ds(sid * spt + r * 16, 16), :], sem))
            for d in descs:
                d.wait()

            @pl.when(sid == _NS - 1)
            def _():
                pltpu.sync_copy(zbuf, accsh.at[pl.ds(chunk, 16), :])

            def zcnt(i, _):
                cntloc[pl.ds(i * _L, _L)] = jnp.zeros((_L,), jnp.float32)
                return _
            lax.fori_loop(0, chunk // _L, zcnt, 0)
            plsc.subcore_barrier()

            def slab(sl, c):
                base = sid * ept + sl * slabsz
                pltpu.sync_copy(dst_hbm.at[pl.ds(base, slabsz)], dstv)
                pltpu.sync_copy(idx_hbm.at[pl.ds(base, slabsz)], idxv)

                def step(i, c):
                    d16 = dstv[pl.ds(i * _L, _L)]
                    x16 = idxv[pl.ds(i * _L, _L)]
                    dl = d16 - lo
                    m = (dl >= 0) & (dl < chunk)
                    mi = jnp.where(m, jnp.full((_L,), 1, jnp.int32),
                                   jnp.zeros((_L,), jnp.int32))
                    cum = jnp.cumsum(mi)
                    offs = jnp.where(m, c + cum - 1, trash)
                    plsc.store_scatter(cidx, [offs], x16)
                    plsc.store_scatter(cdst, [offs >> 7, offs & 127], dl)
                    plsc.addupdate_scatter(cntloc,
                                           [jnp.where(m, dl, chunk)], ones)
                    return c + jnp.sum(mi)
                return lax.fori_loop(0, slabsz // _L, step, c)
            c = lax.fori_loop(0, ept // slabsz, slab, jnp.int32(0))

            iot = lax.broadcasted_iota(jnp.int32, (_L,), 0)
            for p in range(_B // _L):
                pofs = c + p * _L + iot
                cidx[pl.ds(c + p * _L, _L)] = jnp.zeros((_L,), jnp.int32)
                plsc.store_scatter(cdst, [pofs >> 7, pofs & 127],
                                   jnp.full((_L,), chunk, jnp.int32))
            nb = (c + _B - 1) // _B

            def batch(j, _):
                pltpu.async_copy(feat_hbm.at[cidx.at[pl.ds(j * _B, _B)]],
                                 rows, sem).wait()
                pltpu.sync_copy(rows, accsh.at[cdst.at[j]], add=True)
                return _
            lax.fori_loop(0, nb, batch, 0)
            plsc.subcore_barrier()

            pltpu.sync_copy(accsh.at[pl.ds(sid * spt, spt), :],
                            acc_hbm.at[pl.ds(lo + sid * spt, spt), :])
            pltpu.sync_copy(cntloc.at[pl.ds(0, chunk)], cntsh.at[sid])
            plsc.subcore_barrier()

            @pl.when(sid < chunk // 128)
            def _():
                pltpu.sync_copy(cntsh.at[:, pl.ds(sid * 128, 128)], redbuf)

                def red(v, _):
                    t = redbuf[0, pl.ds(v * _L, _L)]
                    for r in range(1, _NS):
                        t = t + redbuf[r, pl.ds(v * _L, _L)]
                    sumv[pl.ds(v * _L, _L)] = t
                    return _
                lax.fori_loop(0, 128 // _L, red, 0)
                pltpu.sync_copy(sumv, cnt_hbm.at[pl.ds(lo + sid * 128, 128)])
            plsc.subcore_barrier()
            return carry

        lax.fori_loop(0, nch, chunk_body, 0)

    def run(table, idx, dst):
        f = pl.kernel(
            body,
            out_type=(
                jax.ShapeDtypeStruct((ndst, _D), jnp.float32),
                jax.ShapeDtypeStruct((ndst,), jnp.float32),
            ),
            mesh=_mesh(),
            compiler_params=_params(),
            scratch_types=[
                pltpu.VMEM_SHARED((chunk + 16, _D), jnp.float32),
                pltpu.VMEM_SHARED((_NS, chunk), jnp.float32),
                pltpu.VMEM((slabsz,), jnp.int32),
                pltpu.VMEM((slabsz,), jnp.int32),
                pltpu.VMEM((ept + _B,), jnp.int32),
                pltpu.VMEM(((ept + _B) // _B, _B), jnp.int32),
                pltpu.VMEM((_B, _D), jnp.float32),
                pltpu.VMEM((chunk + 16,), jnp.float32),
                pltpu.VMEM((_NS, 128), jnp.float32),
                pltpu.VMEM((128,), jnp.float32),
                pltpu.VMEM((16, _D), jnp.float32),
                pltpu.SemaphoreType.DMA,
            ],
            name=name,
        )
        return f(table, idx, dst)

    return run


_aggbig = _make_agg(_E1, _N1, 2048, "sage_aggbig")
_aggsmall = _make_agg(_E0, _N0, 1024, "sage_aggsmall")


def _dense_body(acc_ref, cnt_ref, hs_ref, w_ref, b_ref, o_ref):
    rc = 1.0 / jnp.maximum(cnt_ref[...], 1.0)
    h = jnp.maximum(acc_ref[...] * rc + hs_ref[...], 0.0)
    o_ref[...] = jnp.dot(h, w_ref[...],
                         preferred_element_type=jnp.float32) + b_ref[...]


def _dense(acc, cnt, hself, wt, brow):
    m, d = acc.shape
    dout = wt.shape[1]
    bm = 512
    f = pl.pallas_call(
        _dense_body,
        out_shape=jax.ShapeDtypeStruct((m, dout), jnp.float32),
        grid=(m // bm,),
        in_specs=[
            pl.BlockSpec((bm, d), lambda i: (i, 0)),
            pl.BlockSpec((bm, 1), lambda i: (i, 0)),
            pl.BlockSpec((bm, d), lambda i: (i, 0)),
            pl.BlockSpec((d, dout), lambda i: (0, 0)),
            pl.BlockSpec((1, dout), lambda i: (0, 0)),
        ],
        out_specs=pl.BlockSpec((bm, dout), lambda i: (i, 0)),
    )
    return f(acc, cnt.reshape(m, 1), hself, wt, brow)


def kernel(features, W0, b0, W1, b1, sample0, sample1, sample2,
           block0_src, block0_dst, block1_src, block1_dst):
    wt0 = W0.T
    wt1 = W1.T
    b0r = b0.reshape(1, -1)
    b1r = b1.reshape(1, -1)

    idx1, g1, g0 = _stage(block1_src, sample2, sample1, sample0, features)
    acc1, cnt1 = _aggbig(features, idx1, block1_dst)
    acc0a, cnt0 = _aggsmall(g1, block0_src, block0_dst)

    t1 = _dense(acc1, cnt1, g1, wt0, b0r)
    t0 = _dense(acc0a, cnt0, g0, wt0, b0r)

    acc0b, cnt0b = _aggsmall(t1, block0_src, block0_dst)
    out = _dense(acc0b, cnt0b, t0, wt1, b1r)
    return out


# speedup vs baseline: 2.8294x; 1.0830x over previous
import functools

import jax
import jax.numpy as jnp
from jax import lax
from jax.experimental import pallas as pl
from jax.experimental.pallas import tpu as pltpu
from jax.experimental.pallas import tpu_sc as plsc

_NC, _NS, _L = 2, 16, 16
_NW = _NC * _NS

_D = 128
_N0, _N1, _N2 = 2048, 32768, 524288
_E0, _E1 = 32768, 524288
_B = 128


def _mesh():
    return plsc.VectorSubcoreMesh(core_axis_name="c", subcore_axis_name="s")


def _params():
    return pltpu.CompilerParams(needs_layout_passes=False)


def _zero_zbuf(zbuf):
    z = jnp.zeros((_L,), jnp.float32)
    for r in range(16):
        for v in range(_D // _L):
            zbuf[r, pl.ds(v * _L, _L)] = z


def _stage_body(src_hbm, s2_hbm, s1_hbm, s0_hbm, feat_hbm,
                idx1_hbm, g1_hbm, g0_hbm,
                srcv, idx1v, s1v, s0v, rows_a, rows_b, g0rows, s2sh, sem):
    sid = lax.axis_index("s")
    wid = sid * _NC + lax.axis_index("c")
    epw = _E1 // _NW
    ebase = wid * epw

    @pl.when(sid == 0)
    def _():
        pltpu.sync_copy(s2_hbm, s2sh)
    pltpu.sync_copy(src_hbm.at[pl.ds(ebase, epw)], srcv)
    plsc.subcore_barrier()

    def compose_group(g, _):
        descs = []
        for b in range(8):
            off = (g * 8 + b) * _B
            descs.append(pltpu.async_copy(
                s2sh.at[srcv.at[pl.ds(off, _B)]],
                idx1v.at[pl.ds(off, _B)], sem))
        for d in descs:
            d.wait()
        return _
    lax.fori_loop(0, epw // _B // 8, compose_group, 0)
    pltpu.sync_copy(idx1v, idx1_hbm.at[pl.ds(ebase, epw)])

    npw = _N1 // _NW
    nbase = wid * npw
    pltpu.sync_copy(s1_hbm.at[pl.ds(nbase, npw)], s1v)

    def g1_group(g, _):
        j0 = g * 2
        d0 = pltpu.async_copy(
            feat_hbm.at[s1v.at[pl.ds(j0 * _B, _B)]], rows_a, sem)
        d1 = pltpu.async_copy(
            feat_hbm.at[s1v.at[pl.ds((j0 + 1) * _B, _B)]], rows_b, sem)
        d0.wait()
        pltpu.sync_copy(rows_a, g1_hbm.at[pl.ds(nbase + j0 * _B, _B), :])
        d1.wait()
        pltpu.sync_copy(rows_b, g1_hbm.at[pl.ds(nbase + (j0 + 1) * _B, _B), :])
        return _
    lax.fori_loop(0, npw // _B // 2, g1_group, 0)

    zpw = _N0 // _NW
    zbase = wid * zpw
    pltpu.sync_copy(s0_hbm.at[pl.ds(zbase, zpw)], s0v)
    pltpu.async_copy(feat_hbm.at[s0v], g0rows, sem).wait()
    pltpu.sync_copy(g0rows, g0_hbm.at[pl.ds(zbase, zpw), :])


def _stage(block1_src, sample2, sample1, sample0, features):
    f = pl.kernel(
        _stage_body,
        out_type=(
            jax.ShapeDtypeStruct((_E1,), jnp.int32),
            jax.ShapeDtypeStruct((_N1, _D), jnp.float32),
            jax.ShapeDtypeStruct((_N0, _D), jnp.float32),
        ),
        mesh=_mesh(),
        compiler_params=_params(),
        scratch_types=[
            pltpu.VMEM((_E1 // _NW,), jnp.int32),
            pltpu.VMEM((_E1 // _NW,), jnp.int32),
            pltpu.VMEM((_N1 // _NW,), jnp.int32),
            pltpu.VMEM((_N0 // _NW,), jnp.int32),
            pltpu.VMEM((_B, _D), jnp.float32),
            pltpu.VMEM((_B, _D), jnp.float32),
            pltpu.VMEM((_N0 // _NW, _D), jnp.float32),
            pltpu.VMEM_SHARED((_N2,), jnp.int32),
            pltpu.SemaphoreType.DMA,
        ],
        name="sage_stage",
    )
    return f(block1_src, sample2, sample1, sample0, features)


def _make_agg(e_tot, ndst, chunk, name):
    nch = ndst // chunk // _NC
    ept = e_tot // _NS
    slabsz = min(4096, ept)
    trash = ept + _B - 1
    spt = chunk // _NS

    def body(feat_hbm, idx_hbm, dst_hbm,
             acc_hbm, cnt_hbm,
             accsh, dstv, idxv, cidx, cdst, rows_a, rows_b,
             cntloc, zbuf, sem, sem_a, sem_b):
        cid = lax.axis_index("c")
        sid = lax.axis_index("s")
        ones = jnp.ones((_L,), jnp.float32)
        _zero_zbuf(zbuf)

        def chunk_body(kk, carry):
            lo = (nch * cid + kk) * chunk

            descs = []
            for r in range(spt // 16):
                descs.append(pltpu.async_copy(
                    zbuf, accsh.at[pl.ds(sid * spt + r * 16, 16), :], sem))
            for d in descs:
                d.wait()

            @pl.when(sid == _NS - 1)
            def _():
                pltpu.sync_copy(zbuf, accsh.at[pl.ds(chunk, 16), :])

            def zcnt(i, _):
                cntloc[pl.ds(i * _L, _L)] = jnp.zeros((_L,), jnp.float32)
                return _
            lax.fori_loop(0, chunk // _L, zcnt, 0)
            plsc.subcore_barrier()

            def slab(sl, c):
                base = sid * ept + sl * slabsz
                pltpu.sync_copy(dst_hbm.at[pl.ds(base, slabsz)], dstv)
                pltpu.sync_copy(idx_hbm.at[pl.ds(base, slabsz)], idxv)

                def step(i, c):
                    d16 = dstv[pl.ds(i * _L, _L)]
                    x16 = idxv[pl.ds(i * _L, _L)]
                    dl = d16 - lo
                    m = (dl >= 0) & (dl < chunk)
                    mi = jnp.where(m, jnp.full((_L,), 1, jnp.int32),
                                   jnp.zeros((_L,), jnp.int32))
                    cum = jnp.cumsum(mi)
                    offs = jnp.where(m, c + cum - 1, trash)
                    plsc.store_scatter(cidx, [offs], x16)
                    plsc.store_scatter(cdst, [offs >> 7, offs & 127], dl)
                    plsc.addupdate_scatter(cntloc,
                                           [jnp.where(m, dl, chunk)], ones)
                    return c + cum[_L - 1]
                return lax.fori_loop(0, slabsz // _L, step, c)
            c = lax.fori_loop(0, ept // slabsz, slab, jnp.int32(0))

            iot = lax.broadcasted_iota(jnp.int32, (_L,), 0)
            for p in range(_B // _L):
                pofs = c + p * _L + iot
                cidx[pl.ds(c + p * _L, _L)] = jnp.zeros((_L,), jnp.int32)
                plsc.store_scatter(cdst, [pofs >> 7, pofs & 127],
                                   jnp.full((_L,), chunk, jnp.int32))
            nb = (c + _B - 1) // _B

            def gat(j, buf, s):
                return pltpu.async_copy(
                    feat_hbm.at[cidx.at[pl.ds(j * _B, _B)]], buf, s)

            @pl.when(nb > 0)
            def _():
                pltpu.async_copy(feat_hbm.at[cidx.at[pl.ds(0, _B)]],
                                 rows_a, sem_a)

            def pair(g, carry2):
                j0 = 2 * g

                @pl.when(j0 + 1 < nb)
                def _():
                    gat(j0 + 1, rows_b, sem_b)
                pltpu.make_async_copy(
                    feat_hbm.at[cidx.at[pl.ds(j0 * _B, _B)]],
                    rows_a, sem_a).wait()
                pltpu.sync_copy(rows_a, accsh.at[cdst.at[j0]], add=True)

                @pl.when(j0 + 2 < nb)
                def _():
                    gat(j0 + 2, rows_a, sem_a)

                @pl.when(j0 + 1 < nb)
                def _():
                    pltpu.make_async_copy(
                        feat_hbm.at[cidx.at[pl.ds((j0 + 1) * _B, _B)]],
                        rows_b, sem_b).wait()
                    pltpu.sync_copy(rows_b, accsh.at[cdst.at[j0 + 1]],
                                    add=True)
                return carry2
            lax.fori_loop(0, (nb + 1) // 2, pair, 0)
            plsc.subcore_barrier()

            pltpu.sync_copy(accsh.at[pl.ds(sid * spt, spt), :],
                            acc_hbm.at[pl.ds(lo + sid * spt, spt), :])
            pltpu.sync_copy(cntloc.at[pl.ds(0, chunk)],
                            cnt_hbm.at[sid, pl.ds(lo, chunk)])
            plsc.subcore_barrier()
            return carry

        lax.fori_loop(0, nch, chunk_body, 0)

    def run(table, idx, dst):
        f = pl.kernel(
            body,
            out_type=(
                jax.ShapeDtypeStruct((ndst, _D), jnp.float32),
                jax.ShapeDtypeStruct((_NS, ndst), jnp.float32),
            ),
            mesh=_mesh(),
            compiler_params=_params(),
            scratch_types=[
                pltpu.VMEM_SHARED((chunk + 16, _D), jnp.float32),
                pltpu.VMEM((slabsz,), jnp.int32),
                pltpu.VMEM((slabsz,), jnp.int32),
                pltpu.VMEM((ept + _B,), jnp.int32),
                pltpu.VMEM(((ept + _B) // _B, _B), jnp.int32),
                pltpu.VMEM((_B, _D), jnp.float32),
                pltpu.VMEM((_B, _D), jnp.float32),
                pltpu.VMEM((chunk + 16,), jnp.float32),
                pltpu.VMEM((16, _D), jnp.float32),
                pltpu.SemaphoreType.DMA,
                pltpu.SemaphoreType.DMA,
                pltpu.SemaphoreType.DMA,
            ],
            name=name,
        )
        return f(table, idx, dst)

    return run


_aggbig = _make_agg(_E1, _N1, 2048, "sage_aggbig")
_aggsmall = _make_agg(_E0, _N0, 1024, "sage_aggsmall")


def _dense_body(acc_ref, cnt_ref, hs_ref, w_ref, b_ref, o_ref):
    c = jnp.sum(cnt_ref[...], axis=0)
    rc = 1.0 / jnp.maximum(c, 1.0)
    h = jnp.maximum(acc_ref[...] * rc[:, None] + hs_ref[...], 0.0)
    o_ref[...] = jnp.dot(h, w_ref[...],
                         preferred_element_type=jnp.float32) + b_ref[...]


def _dense(acc, cnt, hself, wt, brow):
    m, d = acc.shape
    dout = wt.shape[1]
    bm = 512
    f = pl.pallas_call(
        _dense_body,
        out_shape=jax.ShapeDtypeStruct((m, dout), jnp.float32),
        grid=(m // bm,),
        in_specs=[
            pl.BlockSpec((bm, d), lambda i: (i, 0)),
            pl.BlockSpec((_NS, bm), lambda i: (0, i)),
            pl.BlockSpec((bm, d), lambda i: (i, 0)),
            pl.BlockSpec((d, dout), lambda i: (0, 0)),
            pl.BlockSpec((1, dout), lambda i: (0, 0)),
        ],
        out_specs=pl.BlockSpec((bm, dout), lambda i: (i, 0)),
    )
    return f(acc, cnt, hself, wt, brow)


def kernel(features, W0, b0, W1, b1, sample0, sample1, sample2,
           block0_src, block0_dst, block1_src, block1_dst):
    wt0 = W0.T
    wt1 = W1.T
    b0r = b0.reshape(1, -1)
    b1r = b1.reshape(1, -1)

    idx1, g1, g0 = _stage(block1_src, sample2, sample1, sample0, features)
    acc1, cnt1 = _aggbig(features, idx1, block1_dst)
    acc0a, cnt0 = _aggsmall(g1, block0_src, block0_dst)

    t1 = _dense(acc1, cnt1, g1, wt0, b0r)
    t0 = _dense(acc0a, cnt0, g0, wt0, b0r)

    acc0b, cnt0b = _aggsmall(t1, block0_src, block0_dst)
    out = _dense(acc0b, cnt0b, t0, wt1, b1r)
    return out
